# baseline (speedup 1.0000x reference)
import functools

import jax
import jax.numpy as jnp
from jax import lax
from jax.experimental import pallas as pl
from jax.experimental.pallas import tpu as pltpu
from jax.experimental.pallas import tpu_sc as plsc

N_ROWS = 10000
NNZ = 320000
OUT_F = 128
LANES = 16
NC = 2
NS = 16
NW = NC * NS
EDGES_PER_W = NNZ // NW
CHUNK = 96
NCH = 108
EPWP = NCH * CHUNK
NBUF = 4
GROUPS = OUT_F // LANES
EGROUPS = CHUNK // LANES
ROWS_PER_TILE = 624
ROWS_REMAINDER = N_ROWS - NS * ROWS_PER_TILE


def _sc_body(ed_h, vd_h, w_h, z_h, out_h,
             e0, e1, e2, e3, v0, v1, v2, v3, r0, r1, r2, r3,
             acc, isem, vsem, gsem, ssem):
    c = lax.axis_index("c")
    s = lax.axis_index("s")
    wid = s * NC + c
    ebufs = (e0, e1, e2, e3)
    vbufs = (v0, v1, v2, v3)
    rows = (r0, r1, r2, r3)

    zlo = s * ROWS_PER_TILE
    pltpu.sync_copy(z_h.at[pl.ds(zlo, ROWS_PER_TILE)], acc.at[pl.ds(zlo, ROWS_PER_TILE)])

    @pl.when(s == NS - 1)
    def _zero_tail():
        tail = NS * ROWS_PER_TILE
        pltpu.sync_copy(z_h.at[pl.ds(tail, ROWS_REMAINDER)],
                        acc.at[pl.ds(tail, ROWS_REMAINDER)])

    plsc.subcore_barrier()

    def start_idx(k, b):
        pltpu.async_copy(ed_h.at[wid, k], ebufs[b], isem.at[b])
        pltpu.async_copy(vd_h.at[wid, k], vbufs[b], vsem.at[b])

    def wait_idx(b):
        pltpu.make_async_copy(ed_h.at[wid, 0], ebufs[b], isem.at[b]).wait()

    def wait_val(b):
        pltpu.make_async_copy(vd_h.at[wid, 0], vbufs[b], vsem.at[b]).wait()

    def start_gather(b):
        pltpu.async_copy(w_h.at[ebufs[b].at[1]], rows[b], gsem.at[b])

    def wait_gather(b):
        pltpu.make_async_copy(w_h.at[ebufs[b].at[1]], rows[b], gsem.at[b]).wait()

    def start_scatter(b):
        pltpu.async_copy(rows[b], acc.at[ebufs[b].at[0]], ssem.at[b], add=True)

    def wait_scatter(b):
        pltpu.make_async_copy(rows[b], acc.at[ebufs[b].at[0]], ssem.at[b]).wait()

    def scale(b):
        rbuf = rows[b]
        vbuf = vbufs[b]

        def group_body(g, carry):
            val16 = vbuf[pl.ds(g * LANES, LANES)]
            for l in range(LANES):
                vs = lax.broadcast(val16[l], (LANES,))
                e = g * LANES + l
                for d in range(GROUPS):
                    sl = pl.ds(d * LANES, LANES)
                    rbuf[e, sl] = rbuf[e, sl] * vs
            return carry

        lax.fori_loop(0, EGROUPS, group_body, 0, unroll=False)

    for j in range(3):
        start_idx(j, j)
    for j in range(2):
        wait_idx(j)
        start_gather(j)

    def chunk_body(k, b):
        wait_gather(b)
        wait_val(b)
        scale(b)
        start_scatter(b)

        b3 = (b + 3) % NBUF

        @pl.when(k >= 1)
        def _drain():
            wait_scatter(b3)

        @pl.when(k + 3 < NCH)
        def _idx():
            start_idx(k + 3, b3)

        b2 = (b + 2) % NBUF

        @pl.when(k + 2 < NCH)
        def _gather():
            wait_idx(b2)
            start_gather(b2)

    def outer(jo, carry):
        for b in range(NBUF):
            chunk_body(jo * NBUF + b, b)
        return carry

    lax.fori_loop(0, NCH // NBUF, outer, 0, unroll=False)
    wait_scatter((NCH - 1) % NBUF)
    plsc.subcore_barrier()

    pltpu.sync_copy(acc.at[pl.ds(zlo, ROWS_PER_TILE)],
                    out_h.at[c, pl.ds(zlo, ROWS_PER_TILE)])

    @pl.when(s == NS - 1)
    def _out_tail():
        tail = NS * ROWS_PER_TILE
        pltpu.sync_copy(acc.at[pl.ds(tail, ROWS_REMAINDER)],
                        out_h.at[c, pl.ds(tail, ROWS_REMAINDER)])


def _combine_body(p_ref, b_ref, o_ref):
    o_ref[...] = p_ref[0] + p_ref[1] + b_ref[...]


def _prep(x, fill):
    x2 = x.reshape(NW, EDGES_PER_W)
    x2 = jnp.pad(x2, ((0, 0), (0, EPWP - EDGES_PER_W)), constant_values=fill)
    return x2.reshape(NW, NCH, CHUNK)


@jax.jit
def _run(row, col, value, weight, bias):
    rowp = _prep(row, 0)
    colp = _prep(col, 0)
    valp = _prep(value, 0.0)
    edata = jnp.stack([rowp, colp], axis=2)
    zeros = jnp.zeros((N_ROWS, OUT_F), jnp.float32)
    mesh = plsc.VectorSubcoreMesh(core_axis_name="c", subcore_axis_name="s")
    ebuf_t = pltpu.VMEM((2, CHUNK), jnp.int32)
    vbuf_t = pltpu.VMEM((CHUNK,), jnp.float32)
    rows_t = pltpu.VMEM((CHUNK, OUT_F), jnp.float32)
    partials = pl.kernel(
        _sc_body,
        out_type=jax.ShapeDtypeStruct((NC, N_ROWS, OUT_F), jnp.float32),
        mesh=mesh,
        scratch_types=[
            ebuf_t, ebuf_t, ebuf_t, ebuf_t,
            vbuf_t, vbuf_t, vbuf_t, vbuf_t,
            rows_t, rows_t, rows_t, rows_t,
            pltpu.VMEM_SHARED((N_ROWS, OUT_F), jnp.float32),
            pltpu.SemaphoreType.DMA((NBUF,)),
            pltpu.SemaphoreType.DMA((NBUF,)),
            pltpu.SemaphoreType.DMA((NBUF,)),
            pltpu.SemaphoreType.DMA((NBUF,)),
        ],
    )(edata, valp, weight, zeros)

    blk = 2000
    out = pl.pallas_call(
        _combine_body,
        grid=(N_ROWS // blk,),
        in_specs=[
            pl.BlockSpec((NC, blk, OUT_F), lambda i: (0, i, 0)),
            pl.BlockSpec((1, OUT_F), lambda i: (0, 0)),
        ],
        out_specs=pl.BlockSpec((blk, OUT_F), lambda i: (i, 0)),
        out_shape=jax.ShapeDtypeStruct((N_ROWS, OUT_F), jnp.float32),
    )(partials, bias.reshape(1, OUT_F))
    return out


def kernel(index, value, n, weight, bias):
    row = index[0].astype(jnp.int32)
    col = index[1].astype(jnp.int32)
    return _run(row, col, value.astype(jnp.float32), weight, bias)


# speedup vs baseline: 1.1227x; 1.1227x over previous
import functools

import jax
import jax.numpy as jnp
from jax import lax
from jax.experimental import pallas as pl
from jax.experimental.pallas import tpu as pltpu
from jax.experimental.pallas import tpu_sc as plsc

N_ROWS = 10000
NNZ = 320000
OUT_F = 128
LANES = 16
NC = 2
NS = 16
NW = NC * NS
EDGES_PER_W = NNZ // NW
CHUNK = 128
NCH = 80
EPWP = NCH * CHUNK
GROUPS = OUT_F // LANES
EGROUPS = CHUNK // LANES
ROWS_PER_TILE = 624
ROWS_REMAINDER = N_ROWS - NS * ROWS_PER_TILE


def _sc_body(idx_h, val_h, w_h, z_h, out_h,
             idxA, valA, rows_v, acc, isem):
    c = lax.axis_index("c")
    s = lax.axis_index("s")
    wid = s * NC + c

    i0 = pltpu.async_copy(idx_h.at[wid], idxA, isem.at[0])
    i1 = pltpu.async_copy(val_h.at[wid], valA, isem.at[1])

    zlo = s * ROWS_PER_TILE
    pltpu.sync_copy(z_h.at[pl.ds(zlo, ROWS_PER_TILE)], acc.at[pl.ds(zlo, ROWS_PER_TILE)])

    @pl.when(s == NS - 1)
    def _zero_tail():
        tail = NS * ROWS_PER_TILE
        pltpu.sync_copy(z_h.at[pl.ds(tail, ROWS_REMAINDER)],
                        acc.at[pl.ds(tail, ROWS_REMAINDER)])

    i0.wait()
    i1.wait()
    plsc.subcore_barrier()

    def chunk_body(k, carry):
        pltpu.sync_copy(w_h.at[idxA.at[k, 1]], rows_v)

        def group_body(g, carry2):
            val16 = valA[k, pl.ds(g * LANES, LANES)]
            for l in range(LANES):
                vs = lax.broadcast(val16[l], (LANES,))
                e = g * LANES + l
                for d in range(GROUPS):
                    sl = pl.ds(d * LANES, LANES)
                    rows_v[e, sl] = rows_v[e, sl] * vs
            return carry2

        lax.fori_loop(0, EGROUPS, group_body, 0, unroll=False)
        pltpu.sync_copy(rows_v, acc.at[idxA.at[k, 0]], add=True)
        return carry

    lax.fori_loop(0, NCH, chunk_body, 0, unroll=False)
    plsc.subcore_barrier()

    pltpu.sync_copy(acc.at[pl.ds(zlo, ROWS_PER_TILE)],
                    out_h.at[c, pl.ds(zlo, ROWS_PER_TILE)])

    @pl.when(s == NS - 1)
    def _out_tail():
        tail = NS * ROWS_PER_TILE
        pltpu.sync_copy(acc.at[pl.ds(tail, ROWS_REMAINDER)],
                        out_h.at[c, pl.ds(tail, ROWS_REMAINDER)])


def _combine_body(p_ref, b_ref, o_ref):
    o_ref[...] = p_ref[0] + p_ref[1] + b_ref[...]


def _prep(x, fill):
    x2 = x.reshape(NW, EDGES_PER_W)
    x2 = jnp.pad(x2, ((0, 0), (0, EPWP - EDGES_PER_W)), constant_values=fill)
    return x2.reshape(NW, NCH, CHUNK)


@jax.jit
def _run(row, col, value, weight, bias):
    rowp = _prep(row, 0)
    colp = _prep(col, 0)
    valp = _prep(value, 0.0)
    idata = jnp.stack([rowp, colp], axis=2)
    zeros = jnp.zeros((N_ROWS, OUT_F), jnp.float32)
    mesh = plsc.VectorSubcoreMesh(core_axis_name="c", subcore_axis_name="s")
    partials = pl.kernel(
        _sc_body,
        out_type=jax.ShapeDtypeStruct((NC, N_ROWS, OUT_F), jnp.float32),
        mesh=mesh,
        scratch_types=[
            pltpu.VMEM((NCH, 2, CHUNK), jnp.int32),
            pltpu.VMEM((NCH, CHUNK), jnp.float32),
            pltpu.VMEM((CHUNK, OUT_F), jnp.float32),
            pltpu.VMEM_SHARED((N_ROWS, OUT_F), jnp.float32),
            pltpu.SemaphoreType.DMA((2,)),
        ],
    )(idata, valp, weight, zeros)

    blk = 2000
    out = pl.pallas_call(
        _combine_body,
        grid=(N_ROWS // blk,),
        in_specs=[
            pl.BlockSpec((NC, blk, OUT_F), lambda i: (0, i, 0)),
            pl.BlockSpec((1, OUT_F), lambda i: (0, 0)),
        ],
        out_specs=pl.BlockSpec((blk, OUT_F), lambda i: (i, 0)),
        out_shape=jax.ShapeDtypeStruct((N_ROWS, OUT_F), jnp.float32),
    )(partials, bias.reshape(1, OUT_F))
    return out


def kernel(index, value, n, weight, bias):
    row = index[0].astype(jnp.int32)
    col = index[1].astype(jnp.int32)
    return _run(row, col, value.astype(jnp.float32), weight, bias)


# speedup vs baseline: 2.3428x; 2.0868x over previous
import functools

import jax
import jax.numpy as jnp
from jax import lax
from jax.experimental import pallas as pl
from jax.experimental.pallas import tpu as pltpu
from jax.experimental.pallas import tpu_sc as plsc

N_ROWS = 10000
NNZ = 320000
OUT_F = 128
LANES = 16
NC = 2
NS = 16
NW = NC * NS
EDGES_PER_W = NNZ // NW
CHUNK = 112
NCH = 90
EPWP = NCH * CHUNK
NRB = 3
NIB = 6
CHWORDS = CHUNK * OUT_F
GROUPS = OUT_F // LANES
EGROUPS = CHUNK // LANES
ROWS_PER_TILE = 624
ROWS_REMAINDER = N_ROWS - NS * ROWS_PER_TILE


def _sc_body(ed_h, vd_h, w_h, z_h, out_h,
             e0, e1, e2, e3, e4, e5, v0, v1, v2, v3, v4, v5,
             r0, r1, r2, acc, isem, vsem, gsem, ssem):
    c = lax.axis_index("c")
    s = lax.axis_index("s")
    wid = s * NC + c
    ebufs = (e0, e1, e2, e3, e4, e5)
    vbufs = (v0, v1, v2, v3, v4, v5)
    rows = (r0, r1, r2)

    zlo = s * ROWS_PER_TILE
    pltpu.sync_copy(z_h.at[pl.ds(zlo, ROWS_PER_TILE)], acc.at[pl.ds(zlo, ROWS_PER_TILE)])

    @pl.when(s == NS - 1)
    def _zero_tail():
        tail = NS * ROWS_PER_TILE
        pltpu.sync_copy(z_h.at[pl.ds(tail, ROWS_REMAINDER)],
                        acc.at[pl.ds(tail, ROWS_REMAINDER)])

    plsc.subcore_barrier()

    def start_idx(k, si):
        pltpu.async_copy(ed_h.at[wid, k], ebufs[si], isem.at[si])
        pltpu.async_copy(vd_h.at[wid, k], vbufs[si], vsem.at[si])

    def start_gather(si, br):
        pltpu.make_async_copy(ed_h.at[wid, 0], ebufs[si], isem.at[si]).wait()
        pltpu.async_copy(w_h.at[ebufs[si].at[1]], rows[br], gsem.at[br])

    def wait_gather(si, br):
        pltpu.make_async_copy(w_h.at[ebufs[si].at[1]], rows[br], gsem.at[br]).wait()

    def start_scatter(si, br):
        pltpu.async_copy(rows[br], acc.at[ebufs[si].at[0]], ssem.at[br], add=True)

    def wait_scatter(si, br):
        pltpu.make_async_copy(rows[br], acc.at[ebufs[si].at[0]], ssem.at[br]).wait()

    def scale(si, br):
        rbuf = rows[br]
        vbuf = vbufs[si]

        def group_body(g, carry):
            val16 = vbuf[pl.ds(g * LANES, LANES)]
            for l in range(LANES):
                vs = lax.broadcast(val16[l], (LANES,))
                e = g * LANES + l
                for d in range(GROUPS):
                    sl = pl.ds(d * LANES, LANES)
                    rbuf[e, sl] = rbuf[e, sl] * vs
            return carry

        lax.fori_loop(0, EGROUPS, group_body, 0, unroll=False)

    for j in range(5):
        start_idx(j, j)
    for j in range(2):
        start_gather(j, j)

    def chunk_body(k, u):
        br = u % NRB
        si = u % NIB
        pltpu.make_async_copy(vd_h.at[wid, 0], vbufs[si], vsem.at[si]).wait()
        wait_gather(si, br)
        scale(si, br)
        start_scatter(si, br)

        b2 = (u + 2) % NRB

        @pl.when(k >= 1)
        def _drain():
            wait_scatter((u + 5) % NIB, b2)

        @pl.when(k + 5 < NCH)
        def _idx():
            start_idx(k + 5, (u + 5) % NIB)

        @pl.when(k + 2 < NCH)
        def _gather():
            start_gather((u + 2) % NIB, b2)

    def outer(jo, carry):
        for u in range(NIB):
            chunk_body(jo * NIB + u, u)
        return carry

    lax.fori_loop(0, NCH // NIB, outer, 0, unroll=False)
    wait_scatter((NCH - 1) % NIB, (NCH - 1) % NRB)
    plsc.subcore_barrier()

    pltpu.sync_copy(acc.at[pl.ds(zlo, ROWS_PER_TILE)],
                    out_h.at[c, pl.ds(zlo, ROWS_PER_TILE)])

    @pl.when(s == NS - 1)
    def _out_tail():
        tail = NS * ROWS_PER_TILE
        pltpu.sync_copy(acc.at[pl.ds(tail, ROWS_REMAINDER)],
                        out_h.at[c, pl.ds(tail, ROWS_REMAINDER)])


def _combine_body(p_ref, b_ref, o_ref):
    o_ref[...] = p_ref[0] + p_ref[1] + b_ref[...]


def _prep(x, fill):
    x2 = x.reshape(NW, EDGES_PER_W)
    x2 = jnp.pad(x2, ((0, 0), (0, EPWP - EDGES_PER_W)), constant_values=fill)
    return x2.reshape(NW, NCH, CHUNK)


@jax.jit
def _run(row, col, value, weight, bias):
    rowp = _prep(row, 0)
    colp = _prep(col, 0)
    valp = _prep(value, 0.0)
    idata = jnp.stack([rowp, colp], axis=2)
    zeros = jnp.zeros((N_ROWS, OUT_F), jnp.float32)
    mesh = plsc.VectorSubcoreMesh(core_axis_name="c", subcore_axis_name="s")
    ebuf_t = pltpu.VMEM((2, CHUNK), jnp.int32)
    vbuf_t = pltpu.VMEM((CHUNK,), jnp.float32)
    rows_t = pltpu.VMEM((CHUNK, OUT_F), jnp.float32)
    partials = pl.kernel(
        _sc_body,
        out_type=jax.ShapeDtypeStruct((NC, N_ROWS, OUT_F), jnp.float32),
        mesh=mesh,
        scratch_types=[
            ebuf_t, ebuf_t, ebuf_t, ebuf_t, ebuf_t, ebuf_t,
            vbuf_t, vbuf_t, vbuf_t, vbuf_t, vbuf_t, vbuf_t,
            rows_t, rows_t, rows_t,
            pltpu.VMEM_SHARED((N_ROWS, OUT_F), jnp.float32),
            pltpu.SemaphoreType.DMA((NIB,)),
            pltpu.SemaphoreType.DMA((NIB,)),
            pltpu.SemaphoreType.DMA((NRB,)),
            pltpu.SemaphoreType.DMA((NRB,)),
        ],
    )(idata, valp, weight, zeros)

    blk = 2000
    out = pl.pallas_call(
        _combine_body,
        grid=(N_ROWS // blk,),
        in_specs=[
            pl.BlockSpec((NC, blk, OUT_F), lambda i: (0, i, 0)),
            pl.BlockSpec((1, OUT_F), lambda i: (0, 0)),
        ],
        out_specs=pl.BlockSpec((blk, OUT_F), lambda i: (i, 0)),
        out_shape=jax.ShapeDtypeStruct((N_ROWS, OUT_F), jnp.float32),
    )(partials, bias.reshape(1, OUT_F))
    return out


def kernel(index, value, n, weight, bias):
    row = index[0].astype(jnp.int32)
    col = index[1].astype(jnp.int32)
    return _run(row, col, value.astype(jnp.float32), weight, bias)


# speedup vs baseline: 2.3590x; 1.0069x over previous
import functools

import jax
import jax.numpy as jnp
from jax import lax
from jax.experimental import pallas as pl
from jax.experimental.pallas import tpu as pltpu
from jax.experimental.pallas import tpu_sc as plsc

N_ROWS = 10000
NNZ = 320000
OUT_F = 128
LANES = 16
NC = 2
NS = 16
NW = NC * NS
EDGES_PER_W = NNZ // NW
CHUNK = 112
NCH = 90
EPWP = NCH * CHUNK
NRB = 3
NIB = 6
CHWORDS = CHUNK * OUT_F
GROUPS = OUT_F // LANES
EGROUPS = CHUNK // LANES
ROWS_PER_TILE = 624
ROWS_REMAINDER = N_ROWS - NS * ROWS_PER_TILE


def _sc_body(ed_h, vd_h, w_h, z_h, out_h,
             e0, e1, e2, e3, e4, e5, v0, v1, v2, v3, v4, v5,
             r0, r1, r2, acc, isem, vsem, gsem, ssem):
    c = lax.axis_index("c")
    s = lax.axis_index("s")
    wid = s * NC + c
    ebufs = (e0, e1, e2, e3, e4, e5)
    vbufs = (v0, v1, v2, v3, v4, v5)
    rows = (r0, r1, r2)

    zlo = s * ROWS_PER_TILE

    def start_idx(k, si):
        pltpu.async_copy(ed_h.at[wid, k], ebufs[si], isem.at[si])
        pltpu.async_copy(vd_h.at[wid, k], vbufs[si], vsem.at[si])

    def start_gather(si, br):
        pltpu.make_async_copy(ed_h.at[wid, 0], ebufs[si], isem.at[si]).wait()
        pltpu.async_copy(w_h.at[ebufs[si].at[1]], rows[br], gsem.at[br])

    def wait_gather(si, br):
        pltpu.make_async_copy(w_h.at[ebufs[si].at[1]], rows[br], gsem.at[br]).wait()

    def start_scatter(si, br):
        pltpu.async_copy(rows[br], acc.at[ebufs[si].at[0]], ssem.at[br], add=True)

    def wait_scatter(si, br):
        pltpu.make_async_copy(rows[br], acc.at[ebufs[si].at[0]], ssem.at[br]).wait()

    def scale(si, br):
        rbuf = rows[br]
        vbuf = vbufs[si]

        def group_body(g, carry):
            val16 = vbuf[pl.ds(g * LANES, LANES)]
            for l in range(LANES):
                vs = lax.broadcast(val16[l], (LANES,))
                e = g * LANES + l
                for d in range(GROUPS):
                    sl = pl.ds(d * LANES, LANES)
                    rbuf[e, sl] = rbuf[e, sl] * vs
            return carry

        lax.fori_loop(0, EGROUPS, group_body, 0, unroll=False)

    for j in range(5):
        start_idx(j, j)
    for j in range(2):
        start_gather(j, j)

    pltpu.sync_copy(z_h.at[pl.ds(0, ROWS_PER_TILE)], acc.at[pl.ds(zlo, ROWS_PER_TILE)])

    @pl.when(s == NS - 1)
    def _zero_tail():
        tail = NS * ROWS_PER_TILE
        pltpu.sync_copy(z_h.at[pl.ds(ROWS_PER_TILE, ROWS_REMAINDER)],
                        acc.at[pl.ds(tail, ROWS_REMAINDER)])

    plsc.subcore_barrier()

    def chunk_body(k, u):
        br = u % NRB
        si = u % NIB
        pltpu.make_async_copy(vd_h.at[wid, 0], vbufs[si], vsem.at[si]).wait()
        wait_gather(si, br)
        scale(si, br)
        start_scatter(si, br)

        b2 = (u + 2) % NRB

        @pl.when(k >= 1)
        def _drain():
            wait_scatter((u + 5) % NIB, b2)

        @pl.when(k + 5 < NCH)
        def _idx():
            start_idx(k + 5, (u + 5) % NIB)

        @pl.when(k + 2 < NCH)
        def _gather():
            start_gather((u + 2) % NIB, b2)

    def outer(jo, carry):
        for u in range(NIB):
            chunk_body(jo * NIB + u, u)
        return carry

    lax.fori_loop(0, NCH // NIB, outer, 0, unroll=False)
    wait_scatter((NCH - 1) % NIB, (NCH - 1) % NRB)
    plsc.subcore_barrier()

    pltpu.sync_copy(acc.at[pl.ds(zlo, ROWS_PER_TILE)],
                    out_h.at[c, pl.ds(zlo, ROWS_PER_TILE)])

    @pl.when(s == NS - 1)
    def _out_tail():
        tail = NS * ROWS_PER_TILE
        pltpu.sync_copy(acc.at[pl.ds(tail, ROWS_REMAINDER)],
                        out_h.at[c, pl.ds(tail, ROWS_REMAINDER)])


def _combine_body(p_ref, b_ref, o_ref):
    o_ref[...] = p_ref[0] + p_ref[1] + b_ref[...]


def _prep(x, fill):
    x2 = x.reshape(NW, EDGES_PER_W)
    x2 = jnp.pad(x2, ((0, 0), (0, EPWP - EDGES_PER_W)), constant_values=fill)
    return x2.reshape(NW, NCH, CHUNK)


@jax.jit
def _run(row, col, value, weight, bias):
    rowp = _prep(row, 0)
    colp = _prep(col, 0)
    valp = _prep(value, 0.0)
    idata = jnp.stack([rowp, colp], axis=2)
    zeros = jnp.zeros((ROWS_PER_TILE + ROWS_REMAINDER, OUT_F), jnp.float32)
    mesh = plsc.VectorSubcoreMesh(core_axis_name="c", subcore_axis_name="s")
    ebuf_t = pltpu.VMEM((2, CHUNK), jnp.int32)
    vbuf_t = pltpu.VMEM((CHUNK,), jnp.float32)
    rows_t = pltpu.VMEM((CHUNK, OUT_F), jnp.float32)
    partials = pl.kernel(
        _sc_body,
        out_type=jax.ShapeDtypeStruct((NC, N_ROWS, OUT_F), jnp.float32),
        mesh=mesh,
        scratch_types=[
            ebuf_t, ebuf_t, ebuf_t, ebuf_t, ebuf_t, ebuf_t,
            vbuf_t, vbuf_t, vbuf_t, vbuf_t, vbuf_t, vbuf_t,
            rows_t, rows_t, rows_t,
            pltpu.VMEM_SHARED((N_ROWS, OUT_F), jnp.float32),
            pltpu.SemaphoreType.DMA((NIB,)),
            pltpu.SemaphoreType.DMA((NIB,)),
            pltpu.SemaphoreType.DMA((NRB,)),
            pltpu.SemaphoreType.DMA((NRB,)),
        ],
    )(idata, valp, weight, zeros)

    blk = 2000
    out = pl.pallas_call(
        _combine_body,
        grid=(N_ROWS // blk,),
        in_specs=[
            pl.BlockSpec((NC, blk, OUT_F), lambda i: (0, i, 0)),
            pl.BlockSpec((1, OUT_F), lambda i: (0, 0)),
        ],
        out_specs=pl.BlockSpec((blk, OUT_F), lambda i: (i, 0)),
        out_shape=jax.ShapeDtypeStruct((N_ROWS, OUT_F), jnp.float32),
    )(partials, bias.reshape(1, OUT_F))
    return out


def kernel(index, value, n, weight, bias):
    row = index[0].astype(jnp.int32)
    col = index[1].astype(jnp.int32)
    return _run(row, col, value.astype(jnp.float32), weight, bias)
